# baseline (speedup 1.0000x reference)
import functools

import jax
import jax.numpy as jnp
from jax import lax
from jax.experimental import pallas as pl
from jax.experimental.pallas import tpu as pltpu
from jax.experimental.pallas import tpu_sc as plsc

N = 10000
E = 320000
C = 128
H = 2
EPS = 1e-16

NS = 16
EPT = E // NS
CH = 80
NCHUNK = EPT // CH
ROWS_A = 624
ROWS_LAST = N - 15 * ROWS_A


def _make_edge_pass(use_ae: bool, head: int):
    mesh = plsc.VectorSubcoreMesh(core_axis_name="c", subcore_axis_name="s",
                                  num_cores=1)
    W = 4 if use_ae else 2

    def body(*refs):
        (pk_h, xl_hh, asx_hh, adx_hh,
         out_u, out_d,
         asrc_v, adst_v,
         pkb_a, srcb_a, dstb_a, exb_a, rowsb_a,
         pkb_b, srcb_b, dstb_b, exb_b, rowsb_b,
         acc, den,
         sl_a, sg_a, ss_a, sl_b, sg_b, ss_b) = refs

        sid = lax.axis_index("s")

        def phase(h, xl_h, asx_h, adx_h):
            z16 = lax.full((16,), 0.0, jnp.float32)
            iz16 = lax.full((16,), 0, jnp.int32)
            lanes = lax.iota(jnp.int32, 16)
            lanesW = lanes * W
            r0 = sid * ROWS_A
            tbase = sid * EPT

            def zrow(j, c):
                for r in range(C // 16):
                    rowsb_a[j, pl.ds(16 * r, 16)] = z16
                return c

            lax.fori_loop(0, CH, zrow, 0)
            for o in range(CH // 16):
                exb_a[pl.ds(16 * o, 16)] = z16
            for k in range(7):
                pltpu.sync_copy(rowsb_a, acc.at[pl.ds(r0 + 80 * k, 80)])
                pltpu.sync_copy(exb_a, den.at[pl.ds(r0 + 80 * k, 80)])

            @pl.when(sid < 15)
            def _():
                pltpu.sync_copy(rowsb_a.at[pl.ds(0, ROWS_A - 560)],
                                acc.at[pl.ds(r0 + 560, ROWS_A - 560)])
                pltpu.sync_copy(exb_a.at[pl.ds(0, ROWS_A - 560 - 16)],
                                den.at[pl.ds(r0 + 560, ROWS_A - 560 - 16)])
                pltpu.sync_copy(exb_a.at[pl.ds(0, 16)],
                                den.at[pl.ds(r0 + ROWS_A - 16, 16)])

            @pl.when(sid == 15)
            def _():
                pltpu.sync_copy(rowsb_a.at[pl.ds(0, ROWS_LAST - 560)],
                                acc.at[pl.ds(r0 + 560, ROWS_LAST - 560)])
                pltpu.sync_copy(exb_a, den.at[pl.ds(r0 + 560, 80)])

            pltpu.sync_copy(asx_h, asrc_v)
            pltpu.sync_copy(adx_h, adst_v)
            plsc.subcore_barrier()

            def lin_d(pkb, sem, g):
                gc = jnp.minimum(g, NCHUNK - 1)
                return pltpu.make_async_copy(
                    pk_h.at[pl.ds((tbase + gc * CH) * W, CH * W)], pkb, sem)

            def gath_d(srcb, rowsb, sem):
                return pltpu.make_async_copy(xl_h.at[srcb], rowsb, sem)

            def scat_d(rowsb, dstb, sem):
                return pltpu.make_async_copy(rowsb, acc.at[dstb], sem)

            def groups(pkb, srcb, dstb, exb):
                for o in range(CH // 16):
                    fo = W * 16 * o
                    sidx = plsc.load_gather(pkb, [lanesW + fo])
                    didx = plsc.load_gather(pkb, [lanesW + (fo + 1)])
                    srcb[pl.ds(16 * o, 16)] = sidx
                    dstb[pl.ds(16 * o, 16)] = didx
                    al = (plsc.load_gather(asrc_v, [sidx])
                          + plsc.load_gather(adst_v, [didx]))
                    if use_ae:
                        al = al + plsc.bitcast(
                            plsc.load_gather(pkb, [lanesW + (fo + 2 + h)]),
                            jnp.float32)
                    al = jnp.where(al >= 0.0, al, al * 0.2)
                    ex = jnp.exp(al)
                    exb[pl.ds(16 * o, 16)] = ex

            def mul(exb, rowsb):
                def mulb(jj, c2):
                    j0 = 2 * jj
                    ev0 = plsc.load_gather(
                        exb, [lax.full((16,), j0, jnp.int32)])
                    ev1 = plsc.load_gather(
                        exb, [lax.full((16,), j0 + 1, jnp.int32)])
                    for r in range(C // 16):
                        rowsb[j0, pl.ds(16 * r, 16)] = (
                            rowsb[j0, pl.ds(16 * r, 16)] * ev0)
                    for r in range(C // 16):
                        rowsb[j0 + 1, pl.ds(16 * r, 16)] = (
                            rowsb[j0 + 1, pl.ds(16 * r, 16)] * ev1)
                    return c2

                lax.fori_loop(0, CH // 2, mulb, 0)

            A = (pkb_a, srcb_a, dstb_a, exb_a, rowsb_a, sl_a, sg_a, ss_a)
            B = (pkb_b, srcb_b, dstb_b, exb_b, rowsb_b, sl_b, sg_b, ss_b)

            lin_d(pkb_a, sl_a, 0).start()
            lin_d(pkb_b, sl_b, 1).start()

            def pair(gg, carry):
                for S, g in ((A, 2 * gg), (B, 2 * gg + 1)):
                    pkb, srcb, dstb, exb, rowsb, sl, sg, ss = S
                    lin_d(pkb, sl, g).wait()

                    @pl.when(gg > 0)
                    def _():
                        scat_d(rowsb, dstb, ss).wait()

                    groups(pkb, srcb, dstb, exb)
                    pltpu.sync_copy(exb, den.at[dstb], add=True)
                    gath_d(srcb, rowsb, sg).start()
                    lin_d(pkb, sl, g + 2).start()
                for S in (A, B):
                    pkb, srcb, dstb, exb, rowsb, sl, sg, ss = S
                    gath_d(srcb, rowsb, sg).wait()
                    mul(exb, rowsb)
                    scat_d(rowsb, dstb, ss).start(add=True)
                return carry

            lax.fori_loop(0, NCHUNK // 2, pair, 0)
            for S in (A, B):
                pkb, srcb, dstb, exb, rowsb, sl, sg, ss = S
                scat_d(rowsb, dstb, ss).wait()
                lin_d(pkb, sl, 0).wait()
            plsc.subcore_barrier()

            for k in range(7):
                rr = r0 + 80 * k
                pltpu.sync_copy(acc.at[pl.ds(rr, 80)], rowsb_a)
                pltpu.sync_copy(rowsb_a, out_u.at[pl.ds(rr, 80)])

            @pl.when(sid < 15)
            def _():
                pltpu.sync_copy(acc.at[pl.ds(r0 + 560, ROWS_A - 560)],
                                rowsb_a.at[pl.ds(0, ROWS_A - 560)])
                pltpu.sync_copy(rowsb_a.at[pl.ds(0, ROWS_A - 560)],
                                out_u.at[pl.ds(r0 + 560, ROWS_A - 560)])

            @pl.when(sid == 15)
            def _():
                pltpu.sync_copy(acc.at[pl.ds(r0 + 560, ROWS_LAST - 560)],
                                rowsb_a.at[pl.ds(0, ROWS_LAST - 560)])
                pltpu.sync_copy(rowsb_a.at[pl.ds(0, ROWS_LAST - 560)],
                                out_u.at[pl.ds(r0 + 560, ROWS_LAST - 560)])

            for k in range(7):
                rr = r0 + 80 * k
                pltpu.sync_copy(den.at[pl.ds(rr, 80)], exb_a)
                pltpu.sync_copy(exb_a, out_d.at[pl.ds(rr, 80)])

            @pl.when(sid < 15)
            def _():
                rr = r0 + 560
                pltpu.sync_copy(den.at[pl.ds(rr, ROWS_A - 560)],
                                exb_a.at[pl.ds(0, ROWS_A - 560)])
                pltpu.sync_copy(exb_a.at[pl.ds(0, ROWS_A - 560)],
                                out_d.at[pl.ds(rr, ROWS_A - 560)])

            @pl.when(sid == 15)
            def _():
                rr = r0 + 560
                pltpu.sync_copy(den.at[pl.ds(rr, 80)], exb_a)
                pltpu.sync_copy(exb_a, out_d.at[pl.ds(rr, 80)])

        phase(head, xl_hh, asx_hh, adx_hh)

    def bufset():
        return [
            pltpu.VMEM((CH * W,), jnp.int32),
            pltpu.VMEM((CH,), jnp.int32),
            pltpu.VMEM((CH,), jnp.int32),
            pltpu.VMEM((CH,), jnp.float32),
            pltpu.VMEM((CH, C), jnp.float32),
        ]

    scratch = ([
        pltpu.VMEM((N,), jnp.float32),
        pltpu.VMEM((N,), jnp.float32),
    ] + bufset() + bufset() + [
        pltpu.VMEM_SHARED((N, C), jnp.float32),
        pltpu.VMEM_SHARED((N,), jnp.float32),
    ] + [pltpu.SemaphoreType.DMA] * 6)
    out_type = [
        jax.ShapeDtypeStruct((N, C), jnp.float32),
        jax.ShapeDtypeStruct((N,), jnp.float32),
    ]
    return pl.kernel(body, mesh=mesh, out_type=out_type,
                     scratch_types=scratch,
                     compiler_params=pltpu.CompilerParams(
                         needs_layout_passes=False))


_edge_pass_ae0 = _make_edge_pass(True, 0)
_edge_pass_ae1 = _make_edge_pass(True, 1)
_edge_pass_noae0 = _make_edge_pass(False, 0)
_edge_pass_noae1 = _make_edge_pass(False, 1)


BE = 16000
BN = 2000


def _a1_body(ea_ref, w1_ref, b1_ref, wc_ref, bc_ref, o_ref):
    t = jnp.maximum(ea_ref[...] @ w1_ref[...] + b1_ref[...], 0.0)
    o_ref[...] = t @ wc_ref[...] + bc_ref[...]


def _edge_mlp(edge_attr, em_W1, em_b1, Wc, bc):
    return pl.pallas_call(
        _a1_body,
        grid=(E // BE,),
        in_specs=[
            pl.BlockSpec((BE, 2), lambda i: (i, 0)),
            pl.BlockSpec((2, C), lambda i: (0, 0)),
            pl.BlockSpec((1, C), lambda i: (0, 0)),
            pl.BlockSpec((C, H), lambda i: (0, 0)),
            pl.BlockSpec((1, H), lambda i: (0, 0)),
        ],
        out_specs=pl.BlockSpec((BE, H), lambda i: (i, 0)),
        out_shape=jax.ShapeDtypeStruct((E, H), jnp.float32),
    )(edge_attr, em_W1, em_b1, Wc, bc)


def _proj_body(x_ref, w0_ref, w1_ref, as_ref, ad_ref,
               xl0_ref, xl1_ref, a4_ref):
    xl0 = x_ref[...] @ w0_ref[...]
    xl1 = x_ref[...] @ w1_ref[...]
    xl0_ref[...] = xl0
    xl1_ref[...] = xl1
    asw = as_ref[...]
    adw = ad_ref[...]
    a4_ref[...] = jnp.concatenate([
        xl0 @ asw[0:1, :].T, xl1 @ asw[1:2, :].T,
        xl0 @ adw[0:1, :].T, xl1 @ adw[1:2, :].T,
    ], axis=1)


def _node_proj(xin, W0, W1, att_src, att_dst):
    din = xin.shape[1]
    return pl.pallas_call(
        _proj_body,
        grid=(N // BN,),
        in_specs=[
            pl.BlockSpec((BN, din), lambda i: (i, 0)),
            pl.BlockSpec((din, C), lambda i: (0, 0)),
            pl.BlockSpec((din, C), lambda i: (0, 0)),
            pl.BlockSpec((H, C), lambda i: (0, 0)),
            pl.BlockSpec((H, C), lambda i: (0, 0)),
        ],
        out_specs=[
            pl.BlockSpec((BN, C), lambda i: (i, 0)),
            pl.BlockSpec((BN, C), lambda i: (i, 0)),
            pl.BlockSpec((BN, 4), lambda i: (i, 0)),
        ],
        out_shape=[
            jax.ShapeDtypeStruct((N, C), jnp.float32),
            jax.ShapeDtypeStruct((N, C), jnp.float32),
            jax.ShapeDtypeStruct((N, 4), jnp.float32),
        ],
    )(xin, W0, W1, att_src, att_dst)


def _hidden_body(u0_ref, u1_ref, dp0_ref, dp1_ref, b_ref, h_ref):
    d0 = dp0_ref[...]
    d1 = dp1_ref[...]
    h_ref[...] = jnp.maximum(
        jnp.concatenate([u0_ref[...] / (d0 + EPS),
                         u1_ref[...] / (d1 + EPS)], axis=1)
        + b_ref[...], 0.0)


def _hidden(u0, u1, dp0, dp1, g1_b):
    return pl.pallas_call(
        _hidden_body,
        grid=(N // BN,),
        in_specs=[
            pl.BlockSpec((BN, C), lambda i: (i, 0)),
            pl.BlockSpec((BN, C), lambda i: (i, 0)),
            pl.BlockSpec((BN, 1), lambda i: (i, 0)),
            pl.BlockSpec((BN, 1), lambda i: (i, 0)),
            pl.BlockSpec((1, H * C), lambda i: (0, 0)),
        ],
        out_specs=pl.BlockSpec((BN, H * C), lambda i: (i, 0)),
        out_shape=jax.ShapeDtypeStruct((N, H * C), jnp.float32),
    )(u0, u1, dp0, dp1, g1_b)


def _final_body(u0_ref, u1_ref, dp0_ref, dp1_ref, gb_ref, w1_ref, b1_ref,
                w2_ref, b2_ref, o_ref):
    d0 = dp0_ref[...]
    d1 = dp1_ref[...]
    out2 = 0.5 * (u0_ref[...] / (d0 + EPS)
                  + u1_ref[...] / (d1 + EPS)) + gb_ref[...]
    t = jnp.maximum(out2 @ w1_ref[...] + b1_ref[...], 0.0)
    o_ref[...] = t @ w2_ref[...] + b2_ref[...]


def _final(u0, u1, dp0, dp1, g2_b, cl_W1, cl_b1, cl_W2, cl_b2):
    ncls = cl_W2.shape[1]
    return pl.pallas_call(
        _final_body,
        grid=(N // BN,),
        in_specs=[
            pl.BlockSpec((BN, C), lambda i: (i, 0)),
            pl.BlockSpec((BN, C), lambda i: (i, 0)),
            pl.BlockSpec((BN, 1), lambda i: (i, 0)),
            pl.BlockSpec((BN, 1), lambda i: (i, 0)),
            pl.BlockSpec((1, C), lambda i: (0, 0)),
            pl.BlockSpec((C, C), lambda i: (0, 0)),
            pl.BlockSpec((1, C), lambda i: (0, 0)),
            pl.BlockSpec((C, ncls), lambda i: (0, 0)),
            pl.BlockSpec((1, ncls), lambda i: (0, 0)),
        ],
        out_specs=pl.BlockSpec((BN, ncls), lambda i: (i, 0)),
        out_shape=jax.ShapeDtypeStruct((N, ncls), jnp.float32),
    )(u0, u1, dp0, dp1, g2_b, cl_W1, cl_b1, cl_W2, cl_b2)


def kernel(x, edge_index, edge_attr, em_W1, em_b1, em_W2, em_b2,
           g1_W, g1_att_src, g1_att_dst, g1_We, g1_att_e, g1_b,
           g2_W, g2_att_src, g2_att_dst, g2_b,
           cl_W1, cl_b1, cl_W2, cl_b2):
    P = jnp.einsum('khc,hc->kh', g1_We.reshape(C, H, C), g1_att_e)
    Wc = em_W2 @ P
    bc = (em_b2 @ P).reshape(1, H)

    ae = _edge_mlp(edge_attr, em_W1, em_b1.reshape(1, C), Wc, bc)
    aeb = jax.lax.bitcast_convert_type(ae, jnp.int32)
    pk1 = jnp.concatenate([edge_index.T, aeb], axis=1).reshape(-1)
    pk2 = edge_index.T.reshape(-1)

    xl0, xl1, a4 = _node_proj(x, g1_W[:, :C], g1_W[:, C:],
                              g1_att_src, g1_att_dst)

    u0, d0 = _edge_pass_ae0(pk1, xl0, a4[:, 0], a4[:, 2])
    u1, d1 = _edge_pass_ae1(pk1, xl1, a4[:, 1], a4[:, 3])

    h = _hidden(u0, u1, d0.reshape(N, 1), d1.reshape(N, 1),
                g1_b.reshape(1, H * C))

    xl20, xl21, a42 = _node_proj(h, g2_W[:, :C], g2_W[:, C:],
                                 g2_att_src, g2_att_dst)

    u20, d20 = _edge_pass_noae0(pk2, xl20, a42[:, 0], a42[:, 2])
    u21, d21 = _edge_pass_noae1(pk2, xl21, a42[:, 1], a42[:, 3])

    return _final(u20, u21, d20.reshape(N, 1), d21.reshape(N, 1),
                  g2_b.reshape(1, C),
                  cl_W1, cl_b1.reshape(1, C), cl_W2, cl_b2.reshape(1, 4))


# speedup vs baseline: 1.4469x; 1.4469x over previous
import jax
import jax.numpy as jnp
from jax import lax
from jax.experimental import pallas as pl
from jax.experimental.pallas import tpu as pltpu
from jax.experimental.pallas import tpu_sc as plsc

N = 10000
E = 320000
C = 128
H = 2
EPS = 1e-16

NS = 16
EPT = E // NS
CH = 80
NCHUNK = EPT // CH
ROWS_A = 624
ROWS_LAST = N - 15 * ROWS_A


def _make_edge_pass(use_ae: bool):
    mesh = plsc.VectorSubcoreMesh(core_axis_name="c", subcore_axis_name="s")
    W = 4 if use_ae else 2

    def body(*refs):
        (pk_h, xl0_h, xl1_h, as0_h, as1_h, ad0_h, ad1_h,
         out_u, out_d,
         asrc_v, adst_v,
         pkb_a, srcb_a, dstb_a, exb_a, rowsb_a,
         pkb_b, srcb_b, dstb_b, exb_b, rowsb_b,
         acc, den,
         sl_a, sg_a, ss_a, sl_b, sg_b, ss_b) = refs

        cid = lax.axis_index("c")
        sid = lax.axis_index("s")

        def phase(h, xl_h, asx_h, adx_h):
            z16 = lax.full((16,), 0.0, jnp.float32)
            lanes = lax.iota(jnp.int32, 16)
            lanesW = lanes * W
            r0 = sid * ROWS_A
            tbase = sid * EPT

            def zrow(j, c):
                for r in range(C // 16):
                    rowsb_a[j, pl.ds(16 * r, 16)] = z16
                return c

            lax.fori_loop(0, CH, zrow, 0)
            for o in range(CH // 16):
                exb_a[pl.ds(16 * o, 16)] = z16
            for k in range(7):
                pltpu.sync_copy(rowsb_a, acc.at[pl.ds(r0 + 80 * k, 80)])
                pltpu.sync_copy(exb_a, den.at[pl.ds(r0 + 80 * k, 80)])

            @pl.when(sid < 15)
            def _():
                pltpu.sync_copy(rowsb_a.at[pl.ds(0, ROWS_A - 560)],
                                acc.at[pl.ds(r0 + 560, ROWS_A - 560)])
                pltpu.sync_copy(exb_a.at[pl.ds(0, ROWS_A - 560 - 16)],
                                den.at[pl.ds(r0 + 560, ROWS_A - 560 - 16)])
                pltpu.sync_copy(exb_a.at[pl.ds(0, 16)],
                                den.at[pl.ds(r0 + ROWS_A - 16, 16)])

            @pl.when(sid == 15)
            def _():
                pltpu.sync_copy(rowsb_a.at[pl.ds(0, ROWS_LAST - 560)],
                                acc.at[pl.ds(r0 + 560, ROWS_LAST - 560)])
                pltpu.sync_copy(exb_a, den.at[pl.ds(r0 + 560, 80)])

            pltpu.sync_copy(asx_h, asrc_v)
            pltpu.sync_copy(adx_h, adst_v)
            plsc.subcore_barrier()

            def lin_d(pkb, sem, g):
                gc = jnp.minimum(g, NCHUNK - 1)
                return pltpu.make_async_copy(
                    pk_h.at[pl.ds((tbase + gc * CH) * W, CH * W)], pkb, sem)

            def gath_d(srcb, rowsb, sem):
                return pltpu.make_async_copy(xl_h.at[srcb], rowsb, sem)

            def scat_d(rowsb, dstb, sem):
                return pltpu.make_async_copy(rowsb, acc.at[dstb], sem)

            def groups(pkb, srcb, dstb, exb):
                for o in range(CH // 16):
                    fo = W * 16 * o
                    sidx = plsc.load_gather(pkb, [lanesW + fo])
                    didx = plsc.load_gather(pkb, [lanesW + (fo + 1)])
                    srcb[pl.ds(16 * o, 16)] = sidx
                    dstb[pl.ds(16 * o, 16)] = didx
                    al = (plsc.load_gather(asrc_v, [sidx])
                          + plsc.load_gather(adst_v, [didx]))
                    if use_ae:
                        al = al + plsc.bitcast(
                            plsc.load_gather(pkb, [lanesW + (fo + 2 + h)]),
                            jnp.float32)
                    al = jnp.where(al >= 0.0, al, al * 0.2)
                    ex = jnp.exp(al)
                    exb[pl.ds(16 * o, 16)] = ex

            def mul(exb, rowsb):
                def mulb(jj, c2):
                    j0 = 2 * jj
                    ev0 = plsc.load_gather(
                        exb, [lax.full((16,), j0, jnp.int32)])
                    ev1 = plsc.load_gather(
                        exb, [lax.full((16,), j0 + 1, jnp.int32)])
                    for r in range(C // 16):
                        rowsb[j0, pl.ds(16 * r, 16)] = (
                            rowsb[j0, pl.ds(16 * r, 16)] * ev0)
                    for r in range(C // 16):
                        rowsb[j0 + 1, pl.ds(16 * r, 16)] = (
                            rowsb[j0 + 1, pl.ds(16 * r, 16)] * ev1)
                    return c2

                lax.fori_loop(0, CH // 2, mulb, 0)

            A = (pkb_a, srcb_a, dstb_a, exb_a, rowsb_a, sl_a, sg_a, ss_a)
            B = (pkb_b, srcb_b, dstb_b, exb_b, rowsb_b, sl_b, sg_b, ss_b)

            lin_d(pkb_a, sl_a, 0).start()
            lin_d(pkb_b, sl_b, 1).start()

            def pair(gg, carry):
                for S, g in ((A, 2 * gg), (B, 2 * gg + 1)):
                    pkb, srcb, dstb, exb, rowsb, sl, sg, ss = S
                    lin_d(pkb, sl, g).wait()

                    @pl.when(gg > 0)
                    def _():
                        scat_d(rowsb, dstb, ss).wait()

                    groups(pkb, srcb, dstb, exb)
                    pltpu.sync_copy(exb, den.at[dstb], add=True)
                    gath_d(srcb, rowsb, sg).start()
                    lin_d(pkb, sl, g + 2).start()
                for S in (A, B):
                    pkb, srcb, dstb, exb, rowsb, sl, sg, ss = S
                    gath_d(srcb, rowsb, sg).wait()
                    mul(exb, rowsb)
                    scat_d(rowsb, dstb, ss).start(add=True)
                return carry

            lax.fori_loop(0, NCHUNK // 2, pair, 0)
            for S in (A, B):
                pkb, srcb, dstb, exb, rowsb, sl, sg, ss = S
                scat_d(rowsb, dstb, ss).wait()
                lin_d(pkb, sl, 0).wait()
            plsc.subcore_barrier()

            for k in range(7):
                rr = r0 + 80 * k
                pltpu.sync_copy(acc.at[pl.ds(rr, 80)], rowsb_a)
                pltpu.sync_copy(rowsb_a, out_u.at[h, pl.ds(rr, 80)])

            @pl.when(sid < 15)
            def _():
                pltpu.sync_copy(acc.at[pl.ds(r0 + 560, ROWS_A - 560)],
                                rowsb_a.at[pl.ds(0, ROWS_A - 560)])
                pltpu.sync_copy(rowsb_a.at[pl.ds(0, ROWS_A - 560)],
                                out_u.at[h, pl.ds(r0 + 560, ROWS_A - 560)])

            @pl.when(sid == 15)
            def _():
                pltpu.sync_copy(acc.at[pl.ds(r0 + 560, ROWS_LAST - 560)],
                                rowsb_a.at[pl.ds(0, ROWS_LAST - 560)])
                pltpu.sync_copy(rowsb_a.at[pl.ds(0, ROWS_LAST - 560)],
                                out_u.at[h, pl.ds(r0 + 560, ROWS_LAST - 560)])

            for k in range(7):
                rr = r0 + 80 * k
                pltpu.sync_copy(den.at[pl.ds(rr, 80)], exb_a)
                pltpu.sync_copy(exb_a, out_d.at[pl.ds(h * N + rr, 80)])

            @pl.when(sid < 15)
            def _():
                rr = r0 + 560
                pltpu.sync_copy(den.at[pl.ds(rr, ROWS_A - 560)],
                                exb_a.at[pl.ds(0, ROWS_A - 560)])
                pltpu.sync_copy(exb_a.at[pl.ds(0, ROWS_A - 560)],
                                out_d.at[pl.ds(h * N + rr, ROWS_A - 560)])

            @pl.when(sid == 15)
            def _():
                rr = r0 + 560
                pltpu.sync_copy(den.at[pl.ds(rr, 80)], exb_a)
                pltpu.sync_copy(exb_a, out_d.at[pl.ds(h * N + rr, 80)])

        @pl.when(cid == 0)
        def _():
            phase(0, xl0_h, as0_h, ad0_h)

        @pl.when(cid == 1)
        def _():
            phase(1, xl1_h, as1_h, ad1_h)

    def bufset():
        return [
            pltpu.VMEM((CH * W,), jnp.int32),
            pltpu.VMEM((CH,), jnp.int32),
            pltpu.VMEM((CH,), jnp.int32),
            pltpu.VMEM((CH,), jnp.float32),
            pltpu.VMEM((CH, C), jnp.float32),
        ]

    scratch = ([
        pltpu.VMEM((N,), jnp.float32),
        pltpu.VMEM((N,), jnp.float32),
    ] + bufset() + bufset() + [
        pltpu.VMEM_SHARED((N, C), jnp.float32),
        pltpu.VMEM_SHARED((N,), jnp.float32),
    ] + [pltpu.SemaphoreType.DMA] * 6)
    out_type = [
        jax.ShapeDtypeStruct((H, N, C), jnp.float32),
        jax.ShapeDtypeStruct((H * N,), jnp.float32),
    ]
    return pl.kernel(body, mesh=mesh, out_type=out_type,
                     scratch_types=scratch,
                     compiler_params=pltpu.CompilerParams(
                         needs_layout_passes=False))


_edge_pass_ae = _make_edge_pass(True)
_edge_pass_noae = _make_edge_pass(False)


BE = 16000
BN = 2000


def _a1_body(ea_ref, w1_ref, b1_ref, wc_ref, bc_ref, o_ref):
    t = jnp.maximum(ea_ref[...] @ w1_ref[...] + b1_ref[...], 0.0)
    o_ref[...] = t @ wc_ref[...] + bc_ref[...]


def _edge_mlp(edge_attr, em_W1, em_b1, Wc, bc):
    return pl.pallas_call(
        _a1_body,
        grid=(E // BE,),
        in_specs=[
            pl.BlockSpec((BE, 2), lambda i: (i, 0)),
            pl.BlockSpec((2, C), lambda i: (0, 0)),
            pl.BlockSpec((1, C), lambda i: (0, 0)),
            pl.BlockSpec((C, H), lambda i: (0, 0)),
            pl.BlockSpec((1, H), lambda i: (0, 0)),
        ],
        out_specs=pl.BlockSpec((BE, H), lambda i: (i, 0)),
        out_shape=jax.ShapeDtypeStruct((E, H), jnp.float32),
    )(edge_attr, em_W1, em_b1, Wc, bc)


def _proj_body(x_ref, w0_ref, w1_ref, as_ref, ad_ref,
               xl0_ref, xl1_ref, a4_ref):
    xl0 = x_ref[...] @ w0_ref[...]
    xl1 = x_ref[...] @ w1_ref[...]
    xl0_ref[...] = xl0
    xl1_ref[...] = xl1
    asw = as_ref[...]
    adw = ad_ref[...]
    a4_ref[...] = jnp.concatenate([
        xl0 @ asw[0:1, :].T, xl1 @ asw[1:2, :].T,
        xl0 @ adw[0:1, :].T, xl1 @ adw[1:2, :].T,
    ], axis=1)


def _node_proj(xin, W0, W1, att_src, att_dst):
    din = xin.shape[1]
    return pl.pallas_call(
        _proj_body,
        grid=(N // BN,),
        in_specs=[
            pl.BlockSpec((BN, din), lambda i: (i, 0)),
            pl.BlockSpec((din, C), lambda i: (0, 0)),
            pl.BlockSpec((din, C), lambda i: (0, 0)),
            pl.BlockSpec((H, C), lambda i: (0, 0)),
            pl.BlockSpec((H, C), lambda i: (0, 0)),
        ],
        out_specs=[
            pl.BlockSpec((BN, C), lambda i: (i, 0)),
            pl.BlockSpec((BN, C), lambda i: (i, 0)),
            pl.BlockSpec((BN, 4), lambda i: (i, 0)),
        ],
        out_shape=[
            jax.ShapeDtypeStruct((N, C), jnp.float32),
            jax.ShapeDtypeStruct((N, C), jnp.float32),
            jax.ShapeDtypeStruct((N, 4), jnp.float32),
        ],
    )(xin, W0, W1, att_src, att_dst)


def _hidden_body(u0_ref, u1_ref, dp0_ref, dp1_ref, b_ref, h_ref):
    d0 = dp0_ref[...]
    d1 = dp1_ref[...]
    h_ref[...] = jnp.maximum(
        jnp.concatenate([u0_ref[...] / (d0 + EPS),
                         u1_ref[...] / (d1 + EPS)], axis=1)
        + b_ref[...], 0.0)


def _hidden(u0, u1, dp0, dp1, g1_b):
    return pl.pallas_call(
        _hidden_body,
        grid=(N // BN,),
        in_specs=[
            pl.BlockSpec((BN, C), lambda i: (i, 0)),
            pl.BlockSpec((BN, C), lambda i: (i, 0)),
            pl.BlockSpec((BN, 1), lambda i: (i, 0)),
            pl.BlockSpec((BN, 1), lambda i: (i, 0)),
            pl.BlockSpec((1, H * C), lambda i: (0, 0)),
        ],
        out_specs=pl.BlockSpec((BN, H * C), lambda i: (i, 0)),
        out_shape=jax.ShapeDtypeStruct((N, H * C), jnp.float32),
    )(u0, u1, dp0, dp1, g1_b)


def _final_body(u0_ref, u1_ref, dp0_ref, dp1_ref, gb_ref, w1_ref, b1_ref,
                w2_ref, b2_ref, o_ref):
    d0 = dp0_ref[...]
    d1 = dp1_ref[...]
    out2 = 0.5 * (u0_ref[...] / (d0 + EPS)
                  + u1_ref[...] / (d1 + EPS)) + gb_ref[...]
    t = jnp.maximum(out2 @ w1_ref[...] + b1_ref[...], 0.0)
    o_ref[...] = t @ w2_ref[...] + b2_ref[...]


def _final(u0, u1, dp0, dp1, g2_b, cl_W1, cl_b1, cl_W2, cl_b2):
    ncls = cl_W2.shape[1]
    return pl.pallas_call(
        _final_body,
        grid=(N // BN,),
        in_specs=[
            pl.BlockSpec((BN, C), lambda i: (i, 0)),
            pl.BlockSpec((BN, C), lambda i: (i, 0)),
            pl.BlockSpec((BN, 1), lambda i: (i, 0)),
            pl.BlockSpec((BN, 1), lambda i: (i, 0)),
            pl.BlockSpec((1, C), lambda i: (0, 0)),
            pl.BlockSpec((C, C), lambda i: (0, 0)),
            pl.BlockSpec((1, C), lambda i: (0, 0)),
            pl.BlockSpec((C, ncls), lambda i: (0, 0)),
            pl.BlockSpec((1, ncls), lambda i: (0, 0)),
        ],
        out_specs=pl.BlockSpec((BN, ncls), lambda i: (i, 0)),
        out_shape=jax.ShapeDtypeStruct((N, ncls), jnp.float32),
    )(u0, u1, dp0, dp1, g2_b, cl_W1, cl_b1, cl_W2, cl_b2)


def kernel(x, edge_index, edge_attr, em_W1, em_b1, em_W2, em_b2,
           g1_W, g1_att_src, g1_att_dst, g1_We, g1_att_e, g1_b,
           g2_W, g2_att_src, g2_att_dst, g2_b,
           cl_W1, cl_b1, cl_W2, cl_b2):
    P = jnp.einsum('khc,hc->kh', g1_We.reshape(C, H, C), g1_att_e)
    Wc = em_W2 @ P
    bc = (em_b2 @ P).reshape(1, H)

    ae = _edge_mlp(edge_attr, em_W1, em_b1.reshape(1, C), Wc, bc)
    aeb = jax.lax.bitcast_convert_type(ae, jnp.int32)
    pk1 = jnp.concatenate([edge_index.T, aeb], axis=1).reshape(-1)
    pk2 = edge_index.T.reshape(-1)

    xl0, xl1, a4 = _node_proj(x, g1_W[:, :C], g1_W[:, C:],
                              g1_att_src, g1_att_dst)

    u, dw = _edge_pass_ae(pk1, xl0, xl1,
                          a4[:, 0], a4[:, 1], a4[:, 2], a4[:, 3])

    h = _hidden(u[0], u[1], dw[:N].reshape(N, 1), dw[N:].reshape(N, 1),
                g1_b.reshape(1, H * C))

    xl20, xl21, a42 = _node_proj(h, g2_W[:, :C], g2_W[:, C:],
                                 g2_att_src, g2_att_dst)

    u2, dw2 = _edge_pass_noae(pk2, xl20, xl21,
                              a42[:, 0], a42[:, 1], a42[:, 2], a42[:, 3])

    return _final(u2[0], u2[1], dw2[:N].reshape(N, 1), dw2[N:].reshape(N, 1),
                  g2_b.reshape(1, C),
                  cl_W1, cl_b1.reshape(1, C), cl_W2, cl_b2.reshape(1, 4))
